# baseline (speedup 1.0000x reference)
import jax
import jax.numpy as jnp
from jax.experimental import pallas as pl

E = 8
T = 2048
H = 2048
TB = 128


def _fused_body(eps_ref, scale_ref, active_ref, token_ref, resid_ref, nw_ref,
                hs_ref, outres_ref):
    e = pl.program_id(1)
    sel = (jax.lax.broadcasted_iota(jnp.int32, (1, E), 1) == e).astype(jnp.float32)
    scol = jnp.sum(scale_ref[0] * sel, axis=1, keepdims=True)
    contrib = active_ref[0] * scol

    @pl.when(e == 0)
    def _init():
        outres_ref[...] = token_ref[...] + resid_ref[...] + contrib

    @pl.when(e != 0)
    def _accum():
        outres_ref[...] = outres_ref[...] + contrib

    @pl.when(e == E - 1)
    def _norm():
        acc = outres_ref[...]
        var = jnp.mean(acc * acc, axis=-1, keepdims=True)
        hs_ref[...] = acc * jax.lax.rsqrt(var + eps_ref[0]) * nw_ref[...]


def kernel(residual, norm_weight, device_num_experts, scale_input,
           active_experts_token_input, token_input, eps):
    del device_num_experts
    eps_arr = jnp.asarray(eps, dtype=jnp.float32).reshape(1)
    nw = norm_weight.reshape(1, H)
    scale_t = scale_input.T.reshape(T // TB, TB, E)

    return pl.pallas_call(
        _fused_body,
        grid=(T // TB, E),
        in_specs=[
            pl.BlockSpec((1,), lambda i, e: (0,)),
            pl.BlockSpec((1, TB, E), lambda i, e: (i, 0, 0)),
            pl.BlockSpec((1, TB, H), lambda i, e: (e, i, 0)),
            pl.BlockSpec((TB, H), lambda i, e: (i, 0)),
            pl.BlockSpec((TB, H), lambda i, e: (i, 0)),
            pl.BlockSpec((1, H), lambda i, e: (0, 0)),
        ],
        out_specs=[
            pl.BlockSpec((TB, H), lambda i, e: (i, 0)),
            pl.BlockSpec((TB, H), lambda i, e: (i, 0)),
        ],
        out_shape=[
            jax.ShapeDtypeStruct((T, H), jnp.float32),
            jax.ShapeDtypeStruct((T, H), jnp.float32),
        ],
    )(eps_arr, scale_t, active_experts_token_input, token_input, residual, nw)


# speedup vs baseline: 1.8574x; 1.8574x over previous
import jax
import jax.numpy as jnp
from jax.experimental import pallas as pl

E = 8
T = 2048
H = 2048
TB = 128


def _fused_body(eps_ref, scale_ref, active_ref, token_ref, resid_ref, nw_ref,
                hs_ref, outres_ref):
    acc = token_ref[...] + resid_ref[...]
    for e in range(E):
        acc = acc + active_ref[e] * scale_ref[0, :, e][:, None]
    outres_ref[...] = acc
    var = jnp.mean(acc * acc, axis=-1, keepdims=True)
    hs_ref[...] = acc * jax.lax.rsqrt(var + eps_ref[0]) * nw_ref[...]


def kernel(residual, norm_weight, device_num_experts, scale_input,
           active_experts_token_input, token_input, eps):
    del device_num_experts
    eps_arr = jnp.asarray(eps, dtype=jnp.float32).reshape(1)
    nw = norm_weight.reshape(1, H)
    scale_t = scale_input.T.reshape(T // TB, TB, E)

    return pl.pallas_call(
        _fused_body,
        grid=(T // TB,),
        in_specs=[
            pl.BlockSpec((1,), lambda i: (0,)),
            pl.BlockSpec((1, TB, E), lambda i: (i, 0, 0)),
            pl.BlockSpec((E, TB, H), lambda i: (0, i, 0)),
            pl.BlockSpec((TB, H), lambda i: (i, 0)),
            pl.BlockSpec((TB, H), lambda i: (i, 0)),
            pl.BlockSpec((1, H), lambda i: (0, 0)),
        ],
        out_specs=[
            pl.BlockSpec((TB, H), lambda i: (i, 0)),
            pl.BlockSpec((TB, H), lambda i: (i, 0)),
        ],
        out_shape=[
            jax.ShapeDtypeStruct((T, H), jnp.float32),
            jax.ShapeDtypeStruct((T, H), jnp.float32),
        ],
    )(eps_arr, scale_t, active_experts_token_input, token_input, residual, nw)
